# Initial kernel scaffold; baseline (speedup 1.0000x reference)
import functools
import math

import jax
import jax.numpy as jnp
from jax import lax
from jax.experimental import pallas as pl
from jax.experimental.pallas import tpu as pltpu
from jax.experimental.pallas import tpu_sc as plsc

_N = 10000
_E = 320000
_D = 128
_H = 128
_R = 2
_RN = _R * _N
_NC = 2
_NS = 16
_RN1 = 20096
_ZR = _RN1 // _NS
_CHUNK = 80
_EPT = _E // (_NC * _NS)
_EPS = _E // _NS

_mesh = plsc.VectorSubcoreMesh(core_axis_name="c", subcore_axis_name="s")


def _deg_body(dst_hbm, typ_hbm, zeros_hbm, ones_hbm, cidx_hbm, deg_hbm,
              dstv, typev, cidxv, onesv, degacc, sem):
    c = lax.axis_index("c")
    s = lax.axis_index("s")
    wid = s * _NC + c
    pltpu.sync_copy(zeros_hbm.at[pl.ds(s * _ZR, _ZR)],
                    degacc.at[pl.ds(s * _ZR, _ZR)])
    pltpu.sync_copy(ones_hbm, onesv)
    plsc.subcore_barrier()
    base0 = wid * _EPT

    @pl.loop(0, _EPT // _CHUNK)
    def _(i):
        base = base0 + i * _CHUNK
        pltpu.sync_copy(dst_hbm.at[pl.ds(base, _CHUNK)], dstv)
        pltpu.sync_copy(typ_hbm.at[pl.ds(base, _CHUNK)], typev)
        for j in range(_CHUNK // 16):
            sl = pl.ds(j * 16, 16)
            cidxv[sl] = dstv[sl] + typev[sl] * _N
        pltpu.sync_copy(cidxv, cidx_hbm.at[pl.ds(base, _CHUNK)])
        pltpu.sync_copy(onesv, degacc.at[cidxv], add=True)

    plsc.subcore_barrier()
    pltpu.sync_copy(degacc.at[pl.ds(s * _ZR, _ZR)],
                    deg_hbm.at[pl.ds(c * _RN1 + s * _ZR, _ZR)])


_deg_kernel = functools.partial(
    pl.kernel,
    out_type=(jax.ShapeDtypeStruct((_E,), jnp.int32),
              jax.ShapeDtypeStruct((_NC * _RN1, 16), jnp.float32)),
    mesh=_mesh,
    scratch_types=[
        pltpu.VMEM((_CHUNK,), jnp.int32),
        pltpu.VMEM((_CHUNK,), jnp.int32),
        pltpu.VMEM((_CHUNK,), jnp.int32),
        pltpu.VMEM((_CHUNK, 16), jnp.float32),
        pltpu.VMEM_SHARED((_RN1, 16), jnp.float32),
        pltpu.SemaphoreType.DMA,
    ],
)(_deg_body)


def _segsum_body(htab_hbm, src_hbm, cidx_hbm, zeros_hbm, agg_hbm,
                 srcv, cidxv, rows, accum, sem):
    c = lax.axis_index("c")
    s = lax.axis_index("s")
    pltpu.sync_copy(zeros_hbm.at[pl.ds(s * _ZR, _ZR)],
                    accum.at[pl.ds(s * _ZR, _ZR)])
    plsc.subcore_barrier()
    base0 = s * _EPS
    coff = c * _N

    @pl.loop(0, _EPS // _CHUNK)
    def _(i):
        base = base0 + i * _CHUNK
        pltpu.sync_copy(src_hbm.at[pl.ds(base, _CHUNK)], srcv)
        for j in range(_CHUNK // 16):
            sl = pl.ds(j * 16, 16)
            srcv[sl] = srcv[sl] + coff
        pltpu.async_copy(htab_hbm.at[srcv], rows, sem).wait()
        pltpu.sync_copy(cidx_hbm.at[pl.ds(base, _CHUNK)], cidxv)
        pltpu.sync_copy(rows, accum.at[cidxv], add=True)

    plsc.subcore_barrier()
    pltpu.sync_copy(accum.at[pl.ds(s * _ZR, _ZR)],
                    agg_hbm.at[pl.ds(c * _RN1 + s * _ZR, _ZR)])


_segsum_kernel = functools.partial(
    pl.kernel,
    out_type=jax.ShapeDtypeStruct((_NC * _RN1, 64), jnp.float32),
    mesh=_mesh,
    scratch_types=[
        pltpu.VMEM((_CHUNK,), jnp.int32),
        pltpu.VMEM((_CHUNK,), jnp.int32),
        pltpu.VMEM((_CHUNK, 64), jnp.float32),
        pltpu.VMEM_SHARED((_RN1, 64), jnp.float32),
        pltpu.SemaphoreType.DMA,
    ],
)(_segsum_body)


def _dgather_body(a_hbm, b_hbm, src_hbm, dst_hbm, outa_hbm, outb_hbm,
                  srcv, dstv, rowsa, rowsb, sema, semb):
    c = lax.axis_index("c")
    s = lax.axis_index("s")
    wid = s * _NC + c
    base0 = wid * _EPT

    @pl.loop(0, _EPT // _CHUNK)
    def _(i):
        base = base0 + i * _CHUNK
        pltpu.sync_copy(src_hbm.at[pl.ds(base, _CHUNK)], srcv)
        pltpu.sync_copy(dst_hbm.at[pl.ds(base, _CHUNK)], dstv)
        ca = pltpu.async_copy(a_hbm.at[srcv], rowsa, sema)
        cb = pltpu.async_copy(b_hbm.at[dstv], rowsb, semb)
        ca.wait()
        cb.wait()
        pltpu.sync_copy(rowsa, outa_hbm.at[pl.ds(base, _CHUNK)])
        pltpu.sync_copy(rowsb, outb_hbm.at[pl.ds(base, _CHUNK)])


_dgather_kernel = functools.partial(
    pl.kernel,
    out_type=(jax.ShapeDtypeStruct((_E, _H), jnp.float32),
              jax.ShapeDtypeStruct((_E, _H), jnp.float32)),
    mesh=_mesh,
    scratch_types=[
        pltpu.VMEM((_CHUNK,), jnp.int32),
        pltpu.VMEM((_CHUNK,), jnp.int32),
        pltpu.VMEM((_CHUNK, _H), jnp.float32),
        pltpu.VMEM((_CHUNK, _H), jnp.float32),
        pltpu.SemaphoreType.DMA,
        pltpu.SemaphoreType.DMA,
    ],
)(_dgather_body)


_BN = 500
_NBLK = _N // _BN
_BNSCALE = 1.0 / math.sqrt(1.0 + 1e-5)


def _dense_common(ht, agg, deg, wroot, b, wrel, g, be, out_ht, wp1, bp1,
                  outa, outb):
    h = jnp.concatenate([ht[0], ht[1]], axis=1)
    acc = jnp.dot(h, wroot[...], preferred_element_type=jnp.float32)
    acc = acc + b[0][None, :]
    for r in range(_R):
        a = jnp.concatenate([agg[0, r], agg[1, r]], axis=1)
        d = deg[0, r, :, 0] + deg[1, r, :, 0]
        inv = 1.0 / jnp.maximum(d, 1.0)
        acc = acc + jnp.dot(a * inv[:, None], wrel[r],
                            preferred_element_type=jnp.float32)
    hn = jnp.maximum(acc * _BNSCALE * g[0][None, :] + be[0][None, :], 0.0)
    out_ht[0] = hn[:, :64]
    out_ht[1] = hn[:, 64:]
    if outa is not None:
        outa[...] = (jnp.dot(hn, wp1[:_H], preferred_element_type=jnp.float32)
                     + bp1[0][None, :])
        outb[...] = jnp.dot(hn, wp1[_H:], preferred_element_type=jnp.float32)


def _dense_body_noab(ht, agg, deg, wroot, b, wrel, g, be, out_ht):
    _dense_common(ht, agg, deg, wroot, b, wrel, g, be, out_ht,
                  None, None, None, None)


def _dense_body_ab(ht, agg, deg, wroot, b, wrel, g, be, wp1, bp1,
                   out_ht, outa, outb):
    _dense_common(ht, agg, deg, wroot, b, wrel, g, be, out_ht, wp1, bp1,
                  outa, outb)


def _make_dense(with_ab):
    in_specs = [
        pl.BlockSpec((2, _BN, 64), lambda i: (0, i, 0)),
        pl.BlockSpec((2, _R, _BN, 64), lambda i: (0, 0, i, 0)),
        pl.BlockSpec((2, _R, _BN, 16), lambda i: (0, 0, i, 0)),
        pl.BlockSpec((_D, _H), lambda i: (0, 0)),
        pl.BlockSpec((1, _H), lambda i: (0, 0)),
        pl.BlockSpec((_R, _D, _H), lambda i: (0, 0, 0)),
        pl.BlockSpec((1, _H), lambda i: (0, 0)),
        pl.BlockSpec((1, _H), lambda i: (0, 0)),
    ]
    out_shape = [jax.ShapeDtypeStruct((2, _N, 64), jnp.float32)]
    out_specs = [pl.BlockSpec((2, _BN, 64), lambda i: (0, i, 0))]
    if with_ab:
        in_specs += [
            pl.BlockSpec((2 * _H, _H), lambda i: (0, 0)),
            pl.BlockSpec((1, _H), lambda i: (0, 0)),
        ]
        out_shape += [jax.ShapeDtypeStruct((_N, _H), jnp.float32),
                      jax.ShapeDtypeStruct((_N, _H), jnp.float32)]
        out_specs += [pl.BlockSpec((_BN, _H), lambda i: (i, 0)),
                      pl.BlockSpec((_BN, _H), lambda i: (i, 0))]
    return pl.pallas_call(
        _dense_body_ab if with_ab else _dense_body_noab,
        grid=(_NBLK,),
        in_specs=in_specs,
        out_specs=out_specs,
        out_shape=out_shape,
    )


_dense_kernel = _make_dense(False)
_dense_ab_kernel = _make_dense(True)


_BE = 512
_EBLK = _E // _BE


def _dec_body(efa, efb, wp2, bp2, wp3, bp3, out):
    t1 = jnp.maximum(efa[...] + efb[...], 0.0)
    t2 = jnp.dot(t1, wp2[...], preferred_element_type=jnp.float32)
    t2 = jnp.maximum(t2 + bp2[0][None, :], 0.0)
    t3 = jnp.sum(t2 * wp3[0][None, :], axis=1) + bp3[0, 0]
    out[...] = t3[None, :]


_dec_kernel = pl.pallas_call(
    _dec_body,
    grid=(_EBLK,),
    in_specs=[
        pl.BlockSpec((_BE, _H), lambda i: (i, 0)),
        pl.BlockSpec((_BE, _H), lambda i: (i, 0)),
        pl.BlockSpec((_H, _H // 2), lambda i: (0, 0)),
        pl.BlockSpec((1, _H // 2), lambda i: (0, 0)),
        pl.BlockSpec((1, _H // 2), lambda i: (0, 0)),
        pl.BlockSpec((1, 1), lambda i: (0, 0)),
    ],
    out_specs=pl.BlockSpec((1, _BE), lambda i: (i, 0)),
    out_shape=jax.ShapeDtypeStruct((_EBLK, _BE), jnp.float32),
)


def kernel(x, edge_index, edge_type, node_emb, Wrel0, Wroot0, b0, g0, be0,
           Wrel1, Wroot1, b1, g1, be1, Wp1, bp1, Wp2, bp2, Wp3, bp3):
    del x
    src = edge_index[0]
    dst = edge_index[1]

    zeros16 = jnp.zeros((_RN1, 16), jnp.float32)
    zeros64 = jnp.zeros((_RN1, 64), jnp.float32)
    ones = jnp.ones((_CHUNK, 16), jnp.float32)

    cidx, deg = _deg_kernel(dst, edge_type, zeros16, ones)
    deg = deg.reshape(_NC, _RN1, 16)[:, :_RN, :].reshape(_NC, _R, _N, 16)

    ht = jnp.stack([node_emb[:, :64], node_emb[:, 64:]])

    agg = _segsum_kernel(ht.reshape(_NC * _N, 64), src, cidx, zeros64)
    agg = agg.reshape(_NC, _RN1, 64)[:, :_RN, :].reshape(_NC, _R, _N, 64)
    (ht,) = _dense_kernel(ht, agg, deg, Wroot0, b0.reshape(1, -1), Wrel0,
                          g0.reshape(1, -1), be0.reshape(1, -1))

    agg = _segsum_kernel(ht.reshape(_NC * _N, 64), src, cidx, zeros64)
    agg = agg.reshape(_NC, _RN1, 64)[:, :_RN, :].reshape(_NC, _R, _N, 64)
    ht, a_tab, b_tab = _dense_ab_kernel(
        ht, agg, deg, Wroot1, b1.reshape(1, -1), Wrel1,
        g1.reshape(1, -1), be1.reshape(1, -1), Wp1, bp1.reshape(1, -1))

    efa, efb = _dgather_kernel(a_tab, b_tab, src, dst)
    out = _dec_kernel(efa, efb, Wp2, bp2.reshape(1, -1),
                      Wp3.reshape(1, -1), bp3.reshape(1, 1))
    return out.reshape(_E)


# speedup vs baseline: 3.0417x; 3.0417x over previous
import functools
import math

import jax
import jax.numpy as jnp
from jax import lax
from jax.experimental import pallas as pl
from jax.experimental.pallas import tpu as pltpu
from jax.experimental.pallas import tpu_sc as plsc

_N = 10000
_E = 320000
_D = 128
_H = 128
_R = 2
_RN = _R * _N
_NC = 2
_NS = 16
_RN1 = 20096
_ZR = _RN1 // _NS
_CHUNK = 80
_EPT = _E // (_NC * _NS)
_EPS = _E // _NS

_mesh = plsc.VectorSubcoreMesh(core_axis_name="c", subcore_axis_name="s")
_sc_params = pltpu.CompilerParams(use_tc_tiling_on_sc=False)


def _deg_body(dst_hbm, typ_hbm, zeros_hbm, ones_hbm, cidx_hbm, deg_hbm,
              dstv, typev, cidxv, onesv, degacc, sem):
    c = lax.axis_index("c")
    s = lax.axis_index("s")
    wid = s * _NC + c
    pltpu.sync_copy(zeros_hbm.at[pl.ds(s * _ZR, _ZR)],
                    degacc.at[pl.ds(s * _ZR, _ZR)])
    pltpu.sync_copy(ones_hbm, onesv)
    plsc.subcore_barrier()
    base0 = wid * _EPT

    @pl.loop(0, _EPT // _CHUNK)
    def _(i):
        base = base0 + i * _CHUNK
        pltpu.sync_copy(dst_hbm.at[pl.ds(base, _CHUNK)], dstv)
        pltpu.sync_copy(typ_hbm.at[pl.ds(base, _CHUNK)], typev)
        for j in range(_CHUNK // 16):
            sl = pl.ds(j * 16, 16)
            cidxv[sl] = dstv[sl] + typev[sl] * _N
        pltpu.sync_copy(cidxv, cidx_hbm.at[pl.ds(base, _CHUNK)])
        pltpu.sync_copy(onesv, degacc.at[cidxv], add=True)

    plsc.subcore_barrier()
    pltpu.sync_copy(degacc.at[pl.ds(s * _ZR, _ZR)],
                    deg_hbm.at[pl.ds(c * _RN1 + s * _ZR, _ZR)])


_deg_kernel = functools.partial(
    pl.kernel,
    out_type=(jax.ShapeDtypeStruct((_E,), jnp.int32),
              jax.ShapeDtypeStruct((_NC * _RN1, 16), jnp.float32)),
    mesh=_mesh,
    compiler_params=_sc_params,
    scratch_types=[
        pltpu.VMEM((_CHUNK,), jnp.int32),
        pltpu.VMEM((_CHUNK,), jnp.int32),
        pltpu.VMEM((_CHUNK,), jnp.int32),
        pltpu.VMEM((_CHUNK, 16), jnp.float32),
        pltpu.VMEM_SHARED((_RN1, 16), jnp.float32),
        pltpu.SemaphoreType.DMA,
    ],
)(_deg_body)


def _segsum_body(htab_hbm, src_hbm, cidx_hbm, zeros_hbm, agg_hbm,
                 srcv, cidxv, rows, accum, sem):
    c = lax.axis_index("c")
    s = lax.axis_index("s")
    pltpu.sync_copy(zeros_hbm.at[pl.ds(s * _ZR, _ZR)],
                    accum.at[pl.ds(s * _ZR, _ZR)])
    plsc.subcore_barrier()
    base0 = s * _EPS
    coff = c * _N

    @pl.loop(0, _EPS // _CHUNK)
    def _(i):
        base = base0 + i * _CHUNK
        pltpu.sync_copy(src_hbm.at[pl.ds(base, _CHUNK)], srcv)
        for j in range(_CHUNK // 16):
            sl = pl.ds(j * 16, 16)
            srcv[sl] = srcv[sl] + coff
        pltpu.async_copy(htab_hbm.at[srcv], rows, sem).wait()
        pltpu.sync_copy(cidx_hbm.at[pl.ds(base, _CHUNK)], cidxv)
        pltpu.sync_copy(rows, accum.at[cidxv], add=True)

    plsc.subcore_barrier()
    pltpu.sync_copy(accum.at[pl.ds(s * _ZR, _ZR)],
                    agg_hbm.at[pl.ds(c * _RN1 + s * _ZR, _ZR)])


_segsum_kernel = functools.partial(
    pl.kernel,
    out_type=jax.ShapeDtypeStruct((_NC * _RN1, 64), jnp.float32),
    mesh=_mesh,
    compiler_params=_sc_params,
    scratch_types=[
        pltpu.VMEM((_CHUNK,), jnp.int32),
        pltpu.VMEM((_CHUNK,), jnp.int32),
        pltpu.VMEM((_CHUNK, 64), jnp.float32),
        pltpu.VMEM_SHARED((_RN1, 64), jnp.float32),
        pltpu.SemaphoreType.DMA,
    ],
)(_segsum_body)


def _dgather_body(a_hbm, b_hbm, src_hbm, dst_hbm, outa_hbm, outb_hbm,
                  srcv, dstv, rowsa, rowsb, sema, semb):
    c = lax.axis_index("c")
    s = lax.axis_index("s")
    wid = s * _NC + c
    base0 = wid * _EPT

    @pl.loop(0, _EPT // _CHUNK)
    def _(i):
        base = base0 + i * _CHUNK
        pltpu.sync_copy(src_hbm.at[pl.ds(base, _CHUNK)], srcv)
        pltpu.sync_copy(dst_hbm.at[pl.ds(base, _CHUNK)], dstv)
        ca = pltpu.async_copy(a_hbm.at[srcv], rowsa, sema)
        cb = pltpu.async_copy(b_hbm.at[dstv], rowsb, semb)
        ca.wait()
        cb.wait()
        pltpu.sync_copy(rowsa, outa_hbm.at[pl.ds(base, _CHUNK)])
        pltpu.sync_copy(rowsb, outb_hbm.at[pl.ds(base, _CHUNK)])


_dgather_kernel = functools.partial(
    pl.kernel,
    out_type=(jax.ShapeDtypeStruct((_E, _H), jnp.float32),
              jax.ShapeDtypeStruct((_E, _H), jnp.float32)),
    mesh=_mesh,
    compiler_params=_sc_params,
    scratch_types=[
        pltpu.VMEM((_CHUNK,), jnp.int32),
        pltpu.VMEM((_CHUNK,), jnp.int32),
        pltpu.VMEM((_CHUNK, _H), jnp.float32),
        pltpu.VMEM((_CHUNK, _H), jnp.float32),
        pltpu.SemaphoreType.DMA,
        pltpu.SemaphoreType.DMA,
    ],
)(_dgather_body)


_BN = 400
_NBLK = _N // _BN
_BNSCALE = 1.0 / math.sqrt(1.0 + 1e-5)


def _dense_common(ht, agg, deg, wroot, b, wrel, g, be, out_ht, wp1, bp1,
                  outa, outb):
    h = jnp.concatenate([ht[0], ht[1]], axis=1)
    acc = jnp.dot(h, wroot[...], preferred_element_type=jnp.float32)
    acc = acc + b[0][None, :]
    for r in range(_R):
        a = jnp.concatenate([agg[0, r], agg[1, r]], axis=1)
        d = deg[0, r, :, 0] + deg[1, r, :, 0]
        inv = 1.0 / jnp.maximum(d, 1.0)
        acc = acc + jnp.dot(a * inv[:, None], wrel[r],
                            preferred_element_type=jnp.float32)
    hn = jnp.maximum(acc * _BNSCALE * g[0][None, :] + be[0][None, :], 0.0)
    out_ht[0] = hn[:, :64]
    out_ht[1] = hn[:, 64:]
    if outa is not None:
        outa[...] = (jnp.dot(hn, wp1[:_H], preferred_element_type=jnp.float32)
                     + bp1[0][None, :])
        outb[...] = jnp.dot(hn, wp1[_H:], preferred_element_type=jnp.float32)


def _dense_body_noab(ht, agg, deg, wroot, b, wrel, g, be, out_ht):
    _dense_common(ht, agg, deg, wroot, b, wrel, g, be, out_ht,
                  None, None, None, None)


def _dense_body_ab(ht, agg, deg, wroot, b, wrel, g, be, wp1, bp1,
                   out_ht, outa, outb):
    _dense_common(ht, agg, deg, wroot, b, wrel, g, be, out_ht, wp1, bp1,
                  outa, outb)


def _make_dense(with_ab):
    in_specs = [
        pl.BlockSpec((2, _BN, 64), lambda i: (0, i, 0)),
        pl.BlockSpec((2, _R, _BN, 64), lambda i: (0, 0, i, 0)),
        pl.BlockSpec((2, _R, _BN, 16), lambda i: (0, 0, i, 0)),
        pl.BlockSpec((_D, _H), lambda i: (0, 0)),
        pl.BlockSpec((1, _H), lambda i: (0, 0)),
        pl.BlockSpec((_R, _D, _H), lambda i: (0, 0, 0)),
        pl.BlockSpec((1, _H), lambda i: (0, 0)),
        pl.BlockSpec((1, _H), lambda i: (0, 0)),
    ]
    out_shape = [jax.ShapeDtypeStruct((2, _N, 64), jnp.float32)]
    out_specs = [pl.BlockSpec((2, _BN, 64), lambda i: (0, i, 0))]
    if with_ab:
        in_specs += [
            pl.BlockSpec((2 * _H, _H), lambda i: (0, 0)),
            pl.BlockSpec((1, _H), lambda i: (0, 0)),
        ]
        out_shape += [jax.ShapeDtypeStruct((_N, _H), jnp.float32),
                      jax.ShapeDtypeStruct((_N, _H), jnp.float32)]
        out_specs += [pl.BlockSpec((_BN, _H), lambda i: (i, 0)),
                      pl.BlockSpec((_BN, _H), lambda i: (i, 0))]
    return pl.pallas_call(
        _dense_body_ab if with_ab else _dense_body_noab,
        grid=(_NBLK,),
        in_specs=in_specs,
        out_specs=out_specs,
        out_shape=out_shape,
    )


_dense_kernel = _make_dense(False)
_dense_ab_kernel = _make_dense(True)


_BE = 512
_EBLK = _E // _BE


def _dec_body(efa, efb, wp2, bp2, wp3, bp3, out):
    t1 = jnp.maximum(efa[...] + efb[...], 0.0)
    t2 = jnp.dot(t1, wp2[...], preferred_element_type=jnp.float32)
    t2 = jnp.maximum(t2 + bp2[0][None, :], 0.0)
    t3 = jnp.sum(t2 * wp3[0][None, :], axis=1) + bp3[0, 0]
    out[0] = t3[None, :]


_dec_kernel = pl.pallas_call(
    _dec_body,
    grid=(_EBLK,),
    in_specs=[
        pl.BlockSpec((_BE, _H), lambda i: (i, 0)),
        pl.BlockSpec((_BE, _H), lambda i: (i, 0)),
        pl.BlockSpec((_H, _H // 2), lambda i: (0, 0)),
        pl.BlockSpec((1, _H // 2), lambda i: (0, 0)),
        pl.BlockSpec((1, _H // 2), lambda i: (0, 0)),
        pl.BlockSpec((1, 1), lambda i: (0, 0)),
    ],
    out_specs=pl.BlockSpec((1, 1, _BE), lambda i: (i, 0, 0)),
    out_shape=jax.ShapeDtypeStruct((_EBLK, 1, _BE), jnp.float32),
)


def kernel(x, edge_index, edge_type, node_emb, Wrel0, Wroot0, b0, g0, be0,
           Wrel1, Wroot1, b1, g1, be1, Wp1, bp1, Wp2, bp2, Wp3, bp3):
    del x
    src = edge_index[0]
    dst = edge_index[1]

    zeros16 = jnp.zeros((_RN1, 16), jnp.float32)
    zeros64 = jnp.zeros((_RN1, 64), jnp.float32)
    ones = jnp.ones((_CHUNK, 16), jnp.float32)

    cidx, deg = _deg_kernel(dst, edge_type, zeros16, ones)
    deg = deg.reshape(_NC, _RN1, 16)[:, :_RN, :].reshape(_NC, _R, _N, 16)

    ht = jnp.stack([node_emb[:, :64], node_emb[:, 64:]])

    agg = _segsum_kernel(ht.reshape(_NC * _N, 64), src, cidx, zeros64)
    agg = agg.reshape(_NC, _RN1, 64)[:, :_RN, :].reshape(_NC, _R, _N, 64)
    (ht,) = _dense_kernel(ht, agg, deg, Wroot0, b0.reshape(1, -1), Wrel0,
                          g0.reshape(1, -1), be0.reshape(1, -1))

    agg = _segsum_kernel(ht.reshape(_NC * _N, 64), src, cidx, zeros64)
    agg = agg.reshape(_NC, _RN1, 64)[:, :_RN, :].reshape(_NC, _R, _N, 64)
    ht, a_tab, b_tab = _dense_ab_kernel(
        ht, agg, deg, Wroot1, b1.reshape(1, -1), Wrel1,
        g1.reshape(1, -1), be1.reshape(1, -1), Wp1, bp1.reshape(1, -1))

    efa, efb = _dgather_kernel(a_tab, b_tab, src, dst)
    out = _dec_kernel(efa, efb, Wp2, bp2.reshape(1, -1),
                      Wp3.reshape(1, -1), bp3.reshape(1, 1))
    return out.reshape(_E)


# speedup vs baseline: 9.1748x; 3.0163x over previous
import functools
import math

import jax
import jax.numpy as jnp
from jax import lax
from jax.experimental import pallas as pl
from jax.experimental.pallas import tpu as pltpu
from jax.experimental.pallas import tpu_sc as plsc

_N = 10000
_E = 320000
_D = 128
_H = 128
_R = 2
_RN = _R * _N
_NC = 2
_NS = 16
_ZR = _RN // _NS
_CHUNK = 80
_NCH = _E // _CHUNK
_TCH = _NCH // (_NC * _NS)
_SCH = _NCH // _NS

_mesh = plsc.VectorSubcoreMesh(core_axis_name="c", subcore_axis_name="s")
_sc_params = pltpu.CompilerParams(use_tc_tiling_on_sc=False)


_G = 50


def _segsum_common(with_deg, htab_hbm, src_hbm, dst_hbm, typ_hbm, zeros_hbm,
                   zeros16_hbm, ones_hbm, agg_hbm, deg_hbm,
                   idxs, idxd, typv, rows0, rows1, onesv, accum, degacc,
                   sem0, sem1):
    c = lax.axis_index("c")
    s = lax.axis_index("s")
    rows = (rows0, rows1)
    sems = (sem0, sem1)
    pltpu.sync_copy(zeros_hbm.at[pl.ds(s * _ZR, _ZR)],
                    accum.at[pl.ds(s * _ZR, _ZR)])
    if with_deg:
        pltpu.sync_copy(zeros16_hbm.at[pl.ds(s * _ZR, _ZR)],
                        degacc.at[pl.ds(s * _ZR, _ZR)])
        pltpu.sync_copy(ones_hbm, onesv)
    cb = s * _SCH
    coff = c
    plsc.subcore_barrier()

    def gath(j, b):
        pltpu.async_copy(htab_hbm.at[idxs.at[j]], rows[b], sems[b])

    def wt(b):
        pltpu.make_async_copy(htab_hbm.at[idxs.at[0]], rows[b],
                              sems[b]).wait()

    @pl.loop(0, _SCH // _G)
    def _(g):
        gb = cb + g * _G
        pltpu.sync_copy(src_hbm.at[pl.ds(gb, _G)], idxs)
        pltpu.sync_copy(dst_hbm.at[pl.ds(gb, _G)], idxd)
        pltpu.sync_copy(typ_hbm.at[pl.ds(gb, _G)], typv)

        @pl.loop(0, _G)
        def _(i):
            for j in range(_CHUNK // 16):
                sl = pl.ds(j * 16, 16)
                idxs[i, sl] = idxs[i, sl] + idxs[i, sl] + coff
                idxd[i, sl] = idxd[i, sl] + typv[i, sl] * _N

        gath(0, 0)
        gath(1, 1)

        @pl.loop(0, _G // 2)
        def _(t):
            for b in range(2):
                jj = t * 2 + b
                wt(b)
                pltpu.sync_copy(rows[b], accum.at[idxd.at[jj]], add=True)
                if with_deg:
                    gchunk = g * _G + jj

                    @pl.when((gchunk < _SCH // 2) == (c == 0))
                    def _():
                        pltpu.sync_copy(onesv, degacc.at[idxd.at[jj]],
                                        add=True)

                @pl.when(jj + 2 < _G)
                def _():
                    gath(jj + 2, b)

    plsc.subcore_barrier()
    pltpu.sync_copy(accum.at[pl.ds(s * _ZR, _ZR)],
                    agg_hbm.at[pl.ds(s * _ZR, _ZR), pl.ds(c * 64, 64)])
    if with_deg:
        pltpu.sync_copy(degacc.at[pl.ds(s * _ZR, _ZR)],
                        deg_hbm.at[pl.ds(s * _ZR, _ZR), pl.ds(c * 16, 16)])


def _segsum_deg_body(htab, src2, dst2, typ2, z64, z16, ones, agg, deg,
                     idxs, idxd, typv, rows0, rows1, onesv, accum, degacc,
                     sem0, sem1):
    _segsum_common(True, htab, src2, dst2, typ2, z64, z16, ones, agg, deg,
                   idxs, idxd, typv, rows0, rows1, onesv, accum, degacc,
                   sem0, sem1)


_segsum_deg_kernel = functools.partial(
    pl.kernel,
    out_type=(jax.ShapeDtypeStruct((_RN, _NC * 64), jnp.float32),
              jax.ShapeDtypeStruct((_RN, _NC * 16), jnp.float32)),
    mesh=_mesh,
    compiler_params=_sc_params,
    scratch_types=[
        pltpu.VMEM((_G, _CHUNK), jnp.int32),
        pltpu.VMEM((_G, _CHUNK), jnp.int32),
        pltpu.VMEM((_G, _CHUNK), jnp.int32),
        pltpu.VMEM((_CHUNK, 64), jnp.float32),
        pltpu.VMEM((_CHUNK, 64), jnp.float32),
        pltpu.VMEM((_CHUNK, 16), jnp.float32),
        pltpu.VMEM_SHARED((_RN, 64), jnp.float32),
        pltpu.VMEM_SHARED((_RN, 16), jnp.float32),
        pltpu.SemaphoreType.DMA,
        pltpu.SemaphoreType.DMA,
    ],
)(_segsum_deg_body)

def _segsum_body(htab, src2, dst2, typ2, z64, agg,
                 idxs, idxd, typv, rows0, rows1, accum, sem0, sem1):
    _segsum_common(False, htab, src2, dst2, typ2, z64, None, None, agg, None,
                   idxs, idxd, typv, rows0, rows1, None, accum, None,
                   sem0, sem1)


_segsum_kernel = functools.partial(
    pl.kernel,
    out_type=jax.ShapeDtypeStruct((_RN, _NC * 64), jnp.float32),
    mesh=_mesh,
    compiler_params=_sc_params,
    scratch_types=[
        pltpu.VMEM((_G, _CHUNK), jnp.int32),
        pltpu.VMEM((_G, _CHUNK), jnp.int32),
        pltpu.VMEM((_G, _CHUNK), jnp.int32),
        pltpu.VMEM((_CHUNK, 64), jnp.float32),
        pltpu.VMEM((_CHUNK, 64), jnp.float32),
        pltpu.VMEM_SHARED((_RN, 64), jnp.float32),
        pltpu.SemaphoreType.DMA,
        pltpu.SemaphoreType.DMA,
    ],
)(_segsum_body)


def _make_dgather(tch, base):
    def body(a_hbm, b_hbm, src_hbm, dst_hbm, out_hbm,
             idxs, idxd, ra0, ra1, rb0, rb1, sa0, sa1, sb0, sb1):
        c = lax.axis_index("c")
        s = lax.axis_index("s")
        wid = s * _NC + c
        ra = (ra0, ra1)
        rb = (rb0, rb1)
        sa = (sa0, sa1)
        sb = (sb0, sb1)
        cb = base + wid * tch
        pltpu.sync_copy(src_hbm.at[pl.ds(cb, tch)], idxs)
        pltpu.sync_copy(dst_hbm.at[pl.ds(cb, tch)], idxd)

        def gath(j, b):
            pltpu.async_copy(a_hbm.at[idxs.at[j]], ra[b], sa[b])
            pltpu.async_copy(b_hbm.at[idxd.at[j]], rb[b], sb[b])

        def wt(b):
            pltpu.make_async_copy(a_hbm.at[idxs.at[0]], ra[b], sa[b]).wait()
            pltpu.make_async_copy(b_hbm.at[idxd.at[0]], rb[b], sb[b]).wait()

        gath(0, 0)
        gath(1, 1)
        ebase = (wid * tch) * _CHUNK

        @pl.loop(0, (tch + 1) // 2)
        def _(t):
            for b in range(2):
                jj = t * 2 + b

                @pl.when(jj < tch)
                def _():
                    wt(b)

                    @pl.loop(0, _CHUNK)
                    def _(i):
                        for j in range(_H // 16):
                            sl = pl.ds(j * 16, 16)
                            ra[b][i, sl] = ra[b][i, sl] + rb[b][i, sl]

                    pltpu.async_copy(
                        ra[b],
                        out_hbm.at[pl.ds(ebase + jj * _CHUNK, _CHUNK)],
                        sa[b]).wait()

                    @pl.when(jj + 2 < tch)
                    def _():
                        gath(jj + 2, b)

    return functools.partial(
        pl.kernel,
        out_type=jax.ShapeDtypeStruct((tch * _NC * _NS * _CHUNK, _H),
                                      jnp.float32),
        mesh=_mesh,
        compiler_params=_sc_params,
        scratch_types=[
            pltpu.VMEM((tch, _CHUNK), jnp.int32),
            pltpu.VMEM((tch, _CHUNK), jnp.int32),
            pltpu.VMEM((_CHUNK, _H), jnp.float32),
            pltpu.VMEM((_CHUNK, _H), jnp.float32),
            pltpu.VMEM((_CHUNK, _H), jnp.float32),
            pltpu.VMEM((_CHUNK, _H), jnp.float32),
            pltpu.SemaphoreType.DMA,
            pltpu.SemaphoreType.DMA,
            pltpu.SemaphoreType.DMA,
            pltpu.SemaphoreType.DMA,
        ],
    )(body)


_TCHS = (32, 32, 31, 30)
_dgathers = []
_base = 0
for _tch in _TCHS:
    _dgathers.append(_make_dgather(_tch, _base))
    _base += _tch * _NC * _NS

_BN = 2000
_NBLK = _N // _BN
_BNSCALE = 1.0 / math.sqrt(1.0 + 1e-5)


def _dense_common(ht, a0, a1, d0, d1, wroot, b, wrel, g, be, out_ht,
                  wp1, bp1, outa, outb):
    h = ht[...].reshape(_BN, _D)
    acc = jnp.dot(h, wroot[...], preferred_element_type=jnp.float32)
    acc = acc + b[0][None, :]
    for r, (ab, db) in enumerate(((a0, d0), (a1, d1))):
        dv = db[...]
        d = dv[:, 0] + dv[:, 16]
        inv = 1.0 / jnp.maximum(d, 1.0)
        acc = acc + jnp.dot(ab[...] * inv[:, None], wrel[r],
                            preferred_element_type=jnp.float32)
    hn = jnp.maximum(acc * _BNSCALE * g[0][None, :] + be[0][None, :], 0.0)
    out_ht[...] = hn.reshape(_BN, _NC, 64)
    if outa is not None:
        outa[...] = (jnp.dot(hn, wp1[:_H], preferred_element_type=jnp.float32)
                     + bp1[0][None, :])
        outb[...] = jnp.dot(hn, wp1[_H:], preferred_element_type=jnp.float32)


def _dense_body_noab(ht, a0, a1, d0, d1, wroot, b, wrel, g, be, out_ht):
    _dense_common(ht, a0, a1, d0, d1, wroot, b, wrel, g, be, out_ht,
                  None, None, None, None)


def _dense_body_ab(ht, a0, a1, d0, d1, wroot, b, wrel, g, be, wp1, bp1,
                   out_ht, outa, outb):
    _dense_common(ht, a0, a1, d0, d1, wroot, b, wrel, g, be, out_ht, wp1,
                  bp1, outa, outb)


def _make_dense(with_ab):
    nb = _N // _BN
    in_specs = [
        pl.BlockSpec((_BN, _NC, 64), lambda i: (i, 0, 0)),
        pl.BlockSpec((_BN, _D), lambda i: (i, 0)),
        pl.BlockSpec((_BN, _D), lambda i, nb=nb: (nb + i, 0)),
        pl.BlockSpec((_BN, 32), lambda i: (i, 0)),
        pl.BlockSpec((_BN, 32), lambda i, nb=nb: (nb + i, 0)),
        pl.BlockSpec((_D, _H), lambda i: (0, 0)),
        pl.BlockSpec((1, _H), lambda i: (0, 0)),
        pl.BlockSpec((_R, _D, _H), lambda i: (0, 0, 0)),
        pl.BlockSpec((1, _H), lambda i: (0, 0)),
        pl.BlockSpec((1, _H), lambda i: (0, 0)),
    ]
    out_shape = [jax.ShapeDtypeStruct((_N, _NC, 64), jnp.float32)]
    out_specs = [pl.BlockSpec((_BN, _NC, 64), lambda i: (i, 0, 0))]
    if with_ab:
        in_specs += [
            pl.BlockSpec((2 * _H, _H), lambda i: (0, 0)),
            pl.BlockSpec((1, _H), lambda i: (0, 0)),
        ]
        out_shape += [jax.ShapeDtypeStruct((_N, _H), jnp.float32),
                      jax.ShapeDtypeStruct((_N, _H), jnp.float32)]
        out_specs += [pl.BlockSpec((_BN, _H), lambda i: (i, 0)),
                      pl.BlockSpec((_BN, _H), lambda i: (i, 0))]
    return pl.pallas_call(
        _dense_body_ab if with_ab else _dense_body_noab,
        grid=(nb,),
        in_specs=in_specs,
        out_specs=out_specs,
        out_shape=out_shape,
    )


_dense_kernel = _make_dense(False)
_dense_ab_kernel = _make_dense(True)


_BE = 2560
_EBLK = _E // _BE


def _dec_body(ef, wp2, bp2, wp3, bp3, out):
    t1 = jnp.maximum(ef[...], 0.0)
    t2 = jnp.dot(t1, wp2[...], preferred_element_type=jnp.float32)
    t2 = jnp.maximum(t2 + bp2[0][None, :], 0.0)
    t3 = jnp.dot(t2, wp3[...], preferred_element_type=jnp.float32)
    out[0] = t3.reshape(1, _BE) + bp3[0, 0]


def _make_dec(nedges):
    nblk = nedges // _BE
    return pl.pallas_call(
        _dec_body,
        grid=(nblk,),
        in_specs=[
            pl.BlockSpec((_BE, _H), lambda i: (i, 0)),
            pl.BlockSpec((_H, _H // 2), lambda i: (0, 0)),
            pl.BlockSpec((1, _H // 2), lambda i: (0, 0)),
            pl.BlockSpec((_H // 2, 1), lambda i: (0, 0)),
            pl.BlockSpec((1, 1), lambda i: (0, 0)),
        ],
        out_specs=pl.BlockSpec((1, 1, _BE), lambda i: (i, 0, 0)),
        out_shape=jax.ShapeDtypeStruct((nblk, 1, _BE), jnp.float32),
    )


_decs = [_make_dec(_tch * _NC * _NS * _CHUNK) for _tch in _TCHS]


def kernel(x, edge_index, edge_type, node_emb, Wrel0, Wroot0, b0, g0, be0,
           Wrel1, Wroot1, b1, g1, be1, Wp1, bp1, Wp2, bp2, Wp3, bp3):
    del x
    src = edge_index[0].reshape(_NCH, _CHUNK)
    dst = edge_index[1].reshape(_NCH, _CHUNK)
    typ = edge_type.reshape(_NCH, _CHUNK)

    zeros16 = jnp.zeros((_RN, 16), jnp.float32)
    zeros64 = jnp.zeros((_RN, 64), jnp.float32)
    ones = jnp.ones((_CHUNK, 16), jnp.float32)

    ht = node_emb.reshape(_NC * _N, 64)

    agg, deg = _segsum_deg_kernel(ht, src, dst, typ, zeros64, zeros16, ones)
    (ht,) = _dense_kernel(ht.reshape(_N, _NC, 64), agg, agg, deg, deg,
                          Wroot0, b0.reshape(1, -1), Wrel0,
                          g0.reshape(1, -1), be0.reshape(1, -1))

    agg = _segsum_kernel(ht.reshape(_NC * _N, 64), src, dst, typ, zeros64)
    ht, a_tab, b_tab = _dense_ab_kernel(
        ht, agg, agg, deg, deg, Wroot1, b1.reshape(1, -1), Wrel1,
        g1.reshape(1, -1), be1.reshape(1, -1), Wp1, bp1.reshape(1, -1))

    wp2b = bp2.reshape(1, -1)
    bp3b = bp3.reshape(1, 1)
    efs = [dg(a_tab, b_tab, src, dst) for dg in _dgathers]
    outs = [dec(ef, Wp2, wp2b, Wp3, bp3b).reshape(-1)
            for dec, ef in zip(_decs, efs)]
    return jnp.concatenate(outs)
